# baseline (speedup 1.0000x reference)
import jax
import jax.numpy as jnp
from jax import lax
from jax.experimental import pallas as pl
from jax.experimental.pallas import tpu as pltpu
from jax.experimental.pallas import tpu_sc as plsc

B = 8
C = 3
HL = 224
HH = 896
NP = 128
P = 32
FLAT = HL * HL
ROWS = FLAT // 128
NROW = B * C * HH
Q = C * P
W = 2 * 128
NW = 32
PPW = (B * NP) // NW


def _topk_body(att_ref, gum_ref, idx_ref, rid_ref, mb_ref, c0_ref):
    s = att_ref[0]
    g = gum_ref[0]
    scores = jnp.log(jnp.maximum(s, 1e-30)) + g
    iota2 = (lax.broadcasted_iota(jnp.int32, (ROWS, 128), 0) * 128
             + lax.broadcasted_iota(jnp.int32, (ROWS, 128), 1))
    laneiota = lax.broadcasted_iota(jnp.int32, (1, 128), 1)
    subiota = lax.broadcasted_iota(jnp.int32, (NP, 1), 0)
    idxrow = jnp.zeros((1, 128), jnp.int32)
    idxcol = jnp.zeros((NP, 1), jnp.int32)
    big = jnp.int32(2 ** 30)
    neg = jnp.float32(-jnp.inf)
    for n in range(NP):
        maxv = jnp.max(scores)
        fp = jnp.min(jnp.where(scores == maxv, iota2, big))
        scores = jnp.where(iota2 == fp, neg, scores)
        idxrow = jnp.where(laneiota == n, fp, idxrow)
        idxcol = jnp.where(subiota == n, fp, idxcol)
    idx_ref[0] = idxrow

    rows = idxcol // HL
    cols = idxcol % HL
    sr = jnp.clip(4 * rows - 16, 0, HH - P)
    sc = jnp.clip(4 * cols - 16, 0, HH - P)
    b = pl.program_id(0)
    q = lax.broadcasted_iota(jnp.int32, (1, Q), 1)
    rid_ref[0] = (3 * b + q // P) * HH + q % P + sr
    c0 = jnp.minimum(sc // 128, (HH - W) // 128) * 128
    iota16 = lax.broadcasted_iota(jnp.int32, (1, 16), 1)
    mb_ref[0] = (sc - c0) + iota16
    c0_ref[0] = c0 + jnp.zeros((1, 16), jnp.int32)


def _sc_body(xh_ref, att_ref, idx_ref, rid_ref, mb_ref, c0_ref,
             out_ref, samp_ref,
             rid_v, mb_v, c0_v, idx_v, att_v, buf_v, patch_v, samp_v, sem):
    w = lax.axis_index("s") * 2 + lax.axis_index("c")
    base_p = w * PPW
    b = w // (NW // B)
    pltpu.sync_copy(rid_ref.at[pl.ds(base_p, PPW)], rid_v)
    pltpu.sync_copy(mb_ref.at[pl.ds(base_p, PPW)], mb_v)
    pltpu.sync_copy(c0_ref.at[pl.ds(base_p, PPW)], c0_v)
    pltpu.sync_copy(idx_ref.at[pl.ds(base_p, PPW)], idx_v)
    pltpu.sync_copy(att_ref.at[b], att_v)

    for h in (0, 16):
        iv = idx_v[pl.ds(h, 16)]
        samp_v[pl.ds(h, 16)] = plsc.load_gather(att_v, [iv])
    pltpu.sync_copy(samp_v, samp_ref.at[pl.ds(base_p, PPW)])

    iota16 = lax.iota(jnp.int32, 16)

    def patch_step(pp, carry):
        ppv = lax.broadcast(pp, (16,))
        c0vec = plsc.load_gather(c0_v, [ppv, iota16])
        c0 = pl.multiple_of(c0vec[0], 128)
        pltpu.async_copy(
            xh_ref.at[rid_v.at[pp], pl.ds(c0, W)], buf_v, sem).wait()
        mvec = plsc.load_gather(mb_v, [ppv, iota16])
        f1 = mvec
        f2 = mvec + 16

        def row_step(q, oaddr):
            qv = lax.broadcast(q, (16,))
            g1 = plsc.load_gather(buf_v, [qv, f1])
            g2 = plsc.load_gather(buf_v, [qv, f2])
            plsc.store_scatter(patch_v, [oaddr], g1)
            plsc.store_scatter(patch_v, [oaddr + 16], g2)
            return oaddr + P

        lax.fori_loop(0, Q, row_step, iota16)
        pltpu.sync_copy(patch_v, out_ref.at[base_p + pp])
        return carry

    lax.fori_loop(0, PPW, patch_step, 0)


@jax.jit
def kernel(x_low, x_high, attention):
    del x_low
    att3 = attention.reshape(B, ROWS, 128)
    u = jax.random.uniform(jax.random.key(42), (B, FLAT),
                           minval=1e-9, maxval=1.0)
    gum3 = (-jnp.log(-jnp.log(u))).reshape(B, ROWS, 128)

    idx, rid, mb, c0 = pl.pallas_call(
        _topk_body,
        grid=(B,),
        in_specs=[
            pl.BlockSpec((1, ROWS, 128), lambda b: (b, 0, 0)),
            pl.BlockSpec((1, ROWS, 128), lambda b: (b, 0, 0)),
        ],
        out_specs=[
            pl.BlockSpec((1, 1, 128), lambda b: (b, 0, 0)),
            pl.BlockSpec((1, NP, Q), lambda b: (b, 0, 0)),
            pl.BlockSpec((1, NP, 16), lambda b: (b, 0, 0)),
            pl.BlockSpec((1, NP, 16), lambda b: (b, 0, 0)),
        ],
        out_shape=[
            jax.ShapeDtypeStruct((B, 1, 128), jnp.int32),
            jax.ShapeDtypeStruct((B, NP, Q), jnp.int32),
            jax.ShapeDtypeStruct((B, NP, 16), jnp.int32),
            jax.ShapeDtypeStruct((B, NP, 16), jnp.int32),
        ],
    )(att3, gum3)

    xh_rows = x_high.reshape(NROW, HH)
    att_flat = attention.reshape(B, FLAT)

    mesh = plsc.VectorSubcoreMesh(core_axis_name="c", subcore_axis_name="s")
    sc_call = pl.kernel(
        _sc_body, mesh=mesh,
        compiler_params=pltpu.CompilerParams(needs_layout_passes=False),
        out_type=[
            jax.ShapeDtypeStruct((B * NP, C * P * P), jnp.float32),
            jax.ShapeDtypeStruct((B * NP,), jnp.float32),
        ],
        scratch_types=[
            pltpu.VMEM((PPW, Q), jnp.int32),
            pltpu.VMEM((PPW, 16), jnp.int32),
            pltpu.VMEM((PPW, 16), jnp.int32),
            pltpu.VMEM((PPW,), jnp.int32),
            pltpu.VMEM((FLAT,), jnp.float32),
            pltpu.VMEM((Q, W), jnp.float32),
            pltpu.VMEM((C * P * P,), jnp.float32),
            pltpu.VMEM((PPW,), jnp.float32),
            pltpu.SemaphoreType.DMA,
        ],
    )
    patches_flat, samp = sc_call(xh_rows, att_flat, idx.reshape(B * NP),
                                 rid.reshape(B * NP, Q),
                                 mb.reshape(B * NP, 16),
                                 c0.reshape(B * NP, 16))
    patches = patches_flat.reshape(B, NP, C, P, P)
    return patches, samp.reshape(B, NP)
